# baseline (speedup 1.0000x reference)
import jax
import jax.numpy as jnp
from jax import lax
from jax.experimental import pallas as pl
from jax.experimental.pallas import tpu as pltpu
from jax.experimental.pallas import tpu_sc as plsc

N_NODES = 10000
D = 128
E_EDGES = 320000
T_TYPES = 4

NC = 2
NS = 16
L = 16
NW = NC * NS

C = 80
EW = E_EDGES // NW
E_PER_TILE = E_EDGES // NS
TN = T_TYPES * N_NODES
GTOT = E_EDGES // C
NK = EW // C
P1R = 2
NK1 = E_PER_TILE // (P1R * C)
NK1P = E_PER_TILE // C
ZB = 2000
ROWS_A = 624
ROWS_LAST = N_NODES - (NS - 1) * ROWS_A


_BN1 = 2000


def _y_body(x_ref, at_ref, ab_ref, yt_ref, yb_ref):
    xb = x_ref[...]
    yt_ref[0] = jnp.dot(xb, at_ref[0], preferred_element_type=jnp.float32)
    yb_ref[0] = jnp.dot(xb, ab_ref[0], preferred_element_type=jnp.float32)


def _y_tables(x, a_top, a_bot):
    grid = (T_TYPES, N_NODES // _BN1)
    return pl.pallas_call(
        _y_body,
        grid=grid,
        in_specs=[
            pl.BlockSpec((_BN1, D), lambda t, i: (i, 0)),
            pl.BlockSpec((1, D, D), lambda t, i: (t, 0, 0)),
            pl.BlockSpec((1, D, D), lambda t, i: (t, 0, 0)),
        ],
        out_specs=[
            pl.BlockSpec((1, _BN1, D), lambda t, i: (t, i, 0)),
            pl.BlockSpec((1, _BN1, D), lambda t, i: (t, i, 0)),
        ],
        out_shape=[
            jax.ShapeDtypeStruct((T_TYPES, N_NODES, D), jnp.float32),
            jax.ShapeDtypeStruct((T_TYPES, N_NODES, D), jnp.float32),
        ],
    )(x, a_top, a_bot)


_BCAST_DNUMS = lax.GatherDimensionNumbers(
    offset_dims=(), collapsed_slice_dims=(0,), start_index_map=(0,))


def _sc_body(idx4, yt, yb, agg_out,
             i4_v, p1_v, t1a_v, t1b_v, g0_v, g1_v, tid_v, dsc_v, nrm_v,
             r0_v, r1_v, ones1_v, zbuf,
             cnt_sh, agg_sh,
             semA0, semA1, semR0, semR1, semS0, semS1, semN0, semN1):
    semA = (semA0, semA1)
    semR = (semR0, semR1)
    semS = (semS0, semS1)
    semN = (semN0, semN1)
    cid = lax.axis_index("c")
    sid = lax.axis_index("s")
    wid = sid * NC + cid

    def _zero16(i, _):
        zbuf[pl.ds(i * L, L)] = jnp.zeros((L,), jnp.float32)
        return 0
    lax.fori_loop(0, ZB // L, _zero16, 0)

    def _ones16(i, _):
        ones1_v[pl.ds(i * L, L)] = jnp.full((L,), 1.0, jnp.float32)
        return 0
    lax.fori_loop(0, C // L, _ones16, 0)

    def _zrow(i, _):
        for j in range(D // L):
            r0_v[0][i, pl.ds(j * L, L)] = jnp.zeros((L,), jnp.float32)
        return 0
    lax.fori_loop(0, C, _zrow, 0)

    @pl.when(sid == 0)
    def _():
        for b in range(TN // ZB):
            pltpu.sync_copy(zbuf, cnt_sh.at[pl.ds(b * ZB, ZB)])

    row0 = sid * ROWS_A

    def _zero_agg_rows(base, nrows):
        off = 0
        while off < nrows:
            n = min(C, nrows - off)
            pltpu.sync_copy(r0_v[0].at[pl.ds(0, n), :],
                            agg_sh.at[pl.ds(base + off, n), :])
            off += n

    @pl.when(sid < NS - 1)
    def _():
        _zero_agg_rows(row0, ROWS_A)

    @pl.when(sid == NS - 1)
    def _():
        _zero_agg_rows(row0, ROWS_LAST)

    plsc.subcore_barrier()

    def _p1_row(k):
        return pl.ds((sid * NK1P + k) * 4 * C, 4 * C)

    def _p1_fire(k, b):
        @pl.when(k < NK1P)
        def _():
            pltpu.async_copy(idx4.at[_p1_row(k)], i4_v[b], semA[b])

    def _p1_proc(k, b):
        @pl.when(k < NK1P)
        def _():
            pltpu.make_async_copy(idx4.at[_p1_row(k)], i4_v[b], semA[b]).wait()

            def _tid16(j, _):
                t16 = i4_v[b][pl.ds(3 * C + j * L, L)]
                d16 = i4_v[b][pl.ds(2 * C + j * L, L)]
                t1a_v[b][pl.ds(j * L, L)] = t16 * N_NODES + d16
                return 0
            lax.fori_loop(0, C // L, _tid16, 0)
            pltpu.sync_copy(ones1_v, cnt_sh.at[t1a_v[b]], add=True)

    _p1_fire(0, 0)
    _p1_fire(1, 1)

    def _p1_pair(i, _):
        k0 = 2 * i
        _p1_proc(k0, 0)
        _p1_fire(k0 + 2, 0)
        _p1_proc(k0 + 1, 1)
        _p1_fire(k0 + 3, 1)
        return 0
    lax.fori_loop(0, (NK1P + 1) // 2, _p1_pair, 0)

    plsc.subcore_barrier()

    def _to_norm(base):
        pltpu.sync_copy(cnt_sh.at[pl.ds(base, ZB)], zbuf)

        def _nrm16(i, _):
            c16 = zbuf[pl.ds(i * L, L)]
            zbuf[pl.ds(i * L, L)] = 1.0 / jnp.maximum(c16, 1.0)
            return 0
        lax.fori_loop(0, ZB // L, _nrm16, 0)
        pltpu.sync_copy(zbuf, cnt_sh.at[pl.ds(base, ZB)])

    _to_norm(sid * ZB)

    @pl.when(sid < TN // ZB - NS)
    def _():
        _to_norm((NS + sid) * ZB)

    plsc.subcore_barrier()

    def _p3_row(k):
        return pl.ds((wid * NK + k) * 4 * C, 4 * C)

    def _p3_fire_idx(k, b):
        @pl.when(k < NK)
        def _():
            pltpu.async_copy(idx4.at[_p3_row(k)], i4_v[b], semA[b])

    def _p3_fire_rows(k, b):
        @pl.when(k < NK)
        def _():
            pltpu.make_async_copy(idx4.at[_p3_row(k)], i4_v[b], semA[b]).wait()

            def _g16(j, _):
                t16 = i4_v[b][pl.ds(3 * C + j * L, L)] * N_NODES
                g0_v[b][pl.ds(j * L, L)] = t16 + i4_v[b][pl.ds(j * L, L)]
                g1_v[b][pl.ds(j * L, L)] = t16 + i4_v[b][pl.ds(C + j * L, L)]
                d16 = i4_v[b][pl.ds(2 * C + j * L, L)]
                dsc_v[b][pl.ds(j * L, L)] = d16
                tid_v[b][pl.ds(j * L, L)] = t16 + d16
                return 0
            lax.fori_loop(0, C // L, _g16, 0)

            pltpu.async_copy(cnt_sh.at[tid_v[b]], nrm_v[b], semN[b])
            pltpu.async_copy(yt.at[g0_v[b]], r0_v[b], semR[b])
            pltpu.async_copy(yb.at[g1_v[b]], r1_v[b], semR[b])

    def _p3_proc(k, b):
        @pl.when(k < NK)
        def _():
            pltpu.make_async_copy(cnt_sh.at[tid_v[b]], nrm_v[b], semN[b]).wait()
            pltpu.make_async_copy(yt.at[g0_v[b]], r0_v[b], semR[b]).wait()
            pltpu.make_async_copy(yb.at[g1_v[b]], r1_v[b], semR[b]).wait()

            def _scale(jj, _):
                nrm16 = nrm_v[b][pl.ds(jj * L, L)]

                def _lane4(i4, _):
                    for u in range(4):
                        i = i4 * 4 + u
                        e = jj * L + i
                        nrm = lax.gather(
                            nrm16, jnp.full((L, 1), i, jnp.int32),
                            _BCAST_DNUMS, slice_sizes=(1,),
                            mode=lax.GatherScatterMode.PROMISE_IN_BOUNDS)
                        for j in range(D // L):
                            sl = pl.ds(j * L, L)
                            r0_v[b][e, sl] = (
                                (r0_v[b][e, sl] + r1_v[b][e, sl]) * nrm)
                    return 0
                lax.fori_loop(0, L // 4, _lane4, 0)
                return 0
            lax.fori_loop(0, C // L, _scale, 0)

            pltpu.async_copy(r0_v[b], agg_sh.at[dsc_v[b]], semS[b], add=True)

    def _p3_wait_scatter(k, b):
        @pl.when((k >= 0) & (k < NK))
        def _():
            pltpu.make_async_copy(r0_v[b], agg_sh.at[dsc_v[b]], semS[b]).wait()

    _p3_fire_idx(0, 0)
    _p3_fire_rows(0, 0)
    _p3_fire_idx(1, 1)

    def _p3_pair(i, _):
        k0 = 2 * i
        _p3_wait_scatter(k0 - 1, 1)
        _p3_fire_rows(k0 + 1, 1)
        _p3_proc(k0, 0)
        _p3_fire_idx(k0 + 2, 0)
        _p3_proc(k0 + 1, 1)
        _p3_fire_idx(k0 + 3, 1)
        _p3_wait_scatter(k0, 0)
        _p3_fire_rows(k0 + 2, 0)
        return 0
    lax.fori_loop(0, (NK + 1) // 2, _p3_pair, 0)

    plsc.subcore_barrier()

    @pl.when(sid < NS - 1)
    def _():
        pltpu.sync_copy(agg_sh.at[pl.ds(row0, ROWS_A), :],
                        agg_out.at[cid, pl.ds(row0, ROWS_A), :])

    @pl.when(sid == NS - 1)
    def _():
        pltpu.sync_copy(agg_sh.at[pl.ds(row0, ROWS_LAST), :],
                        agg_out.at[cid, pl.ds(row0, ROWS_LAST), :])


def _sc_scatter(idx4, yt, yb):
    mesh = plsc.VectorSubcoreMesh(core_axis_name="c", subcore_axis_name="s",
                                  num_cores=NC, num_subcores=NS)
    f = pl.kernel(
        _sc_body,
        out_type=jax.ShapeDtypeStruct((NC, N_NODES, D), jnp.float32),
        mesh=mesh,
        scratch_types=[
            [pltpu.VMEM((4 * C,), jnp.int32)] * 2,
            [pltpu.VMEM((P1R * 4 * C,), jnp.int32)] * 2,
            [pltpu.VMEM((C,), jnp.int32)] * 2,
            [pltpu.VMEM((C,), jnp.int32)] * 2,
            [pltpu.VMEM((C,), jnp.int32)] * 2,
            [pltpu.VMEM((C,), jnp.int32)] * 2,
            [pltpu.VMEM((C,), jnp.int32)] * 2,
            [pltpu.VMEM((C,), jnp.int32)] * 2,
            [pltpu.VMEM((C,), jnp.float32)] * 2,
            [pltpu.VMEM((C, D), jnp.float32)] * 2,
            [pltpu.VMEM((C, D), jnp.float32)] * 2,
            pltpu.VMEM((C,), jnp.float32),
            pltpu.VMEM((ZB,), jnp.float32),
            pltpu.VMEM_SHARED((TN,), jnp.float32),
            pltpu.VMEM_SHARED((N_NODES, D), jnp.float32),
            pltpu.SemaphoreType.DMA,
            pltpu.SemaphoreType.DMA,
            pltpu.SemaphoreType.DMA,
            pltpu.SemaphoreType.DMA,
            pltpu.SemaphoreType.DMA,
            pltpu.SemaphoreType.DMA,
            pltpu.SemaphoreType.DMA,
            pltpu.SemaphoreType.DMA,
        ],
    )
    return f(idx4, yt, yb)


_BN2 = 2000


def _out_body(x_ref, cwt_ref, cb_ref, a0_ref, a1_ref, o_ref):
    o_ref[...] = (jnp.dot(x_ref[...], cwt_ref[...],
                          preferred_element_type=jnp.float32)
                  + cb_ref[...] + a0_ref[...] + a1_ref[...])


def _combine(x, cwt, cb, a0, a1):
    grid = (N_NODES // _BN2,)
    return pl.pallas_call(
        _out_body,
        grid=grid,
        in_specs=[
            pl.BlockSpec((_BN2, D), lambda i: (i, 0)),
            pl.BlockSpec((D, D), lambda i: (0, 0)),
            pl.BlockSpec((1, D), lambda i: (0, 0)),
            pl.BlockSpec((_BN2, D), lambda i: (i, 0)),
            pl.BlockSpec((_BN2, D), lambda i: (i, 0)),
        ],
        out_specs=pl.BlockSpec((_BN2, D), lambda i: (i, 0)),
        out_shape=jax.ShapeDtypeStruct((N_NODES, D), jnp.float32),
    )(x, cwt, cb, a0, a1)


@jax.jit
def kernel(x, hyperedge_index_2, hyperedge_type_2, A_2, C_w, C_b):
    src = hyperedge_index_2[0]
    src0 = src[0::2]
    src1 = src[1::2]
    dst = hyperedge_index_2[1][0::2]
    a_top = A_2[:, :D, :]
    a_bot = A_2[:, D:, :]

    idx4 = jnp.stack([src0, src1, dst, hyperedge_type_2])
    idx4 = idx4.reshape(4, GTOT, C).transpose(1, 0, 2).reshape(-1)

    yt, yb = _y_tables(x, a_top, a_bot)
    yt = yt.reshape(T_TYPES * N_NODES, D)
    yb = yb.reshape(T_TYPES * N_NODES, D)

    agg = _sc_scatter(idx4, yt, yb)

    return _combine(x, C_w.T, C_b.reshape(1, D), agg[0], agg[1])


# speedup vs baseline: 1.0906x; 1.0906x over previous
import jax
import jax.numpy as jnp
from jax import lax
from jax.experimental import pallas as pl
from jax.experimental.pallas import tpu as pltpu
from jax.experimental.pallas import tpu_sc as plsc

N_NODES = 10000
D = 128
E_EDGES = 320000
T_TYPES = 4

NC = 2
NS = 16
L = 16
NW = NC * NS

C = 80
EW = E_EDGES // NW
E_PER_TILE = E_EDGES // NS
TN = T_TYPES * N_NODES
GTOT = E_EDGES // C
NK = EW // C
P1R = 2
NK1 = E_PER_TILE // (P1R * C)
NK1P = E_PER_TILE // C
ZB = 2000
ROWS_A = 624
ROWS_LAST = N_NODES - (NS - 1) * ROWS_A


_BN1 = 2000


def _y_body(x_ref, at_ref, ab_ref, yt_ref, yb_ref):
    xb = x_ref[...]
    yt_ref[0] = jnp.dot(xb, at_ref[0], preferred_element_type=jnp.float32)
    yb_ref[0] = jnp.dot(xb, ab_ref[0], preferred_element_type=jnp.float32)


def _y_tables(x, a_top, a_bot):
    grid = (T_TYPES, N_NODES // _BN1)
    return pl.pallas_call(
        _y_body,
        grid=grid,
        in_specs=[
            pl.BlockSpec((_BN1, D), lambda t, i: (i, 0)),
            pl.BlockSpec((1, D, D), lambda t, i: (t, 0, 0)),
            pl.BlockSpec((1, D, D), lambda t, i: (t, 0, 0)),
        ],
        out_specs=[
            pl.BlockSpec((1, _BN1, D), lambda t, i: (t, i, 0)),
            pl.BlockSpec((1, _BN1, D), lambda t, i: (t, i, 0)),
        ],
        out_shape=[
            jax.ShapeDtypeStruct((T_TYPES, N_NODES, D), jnp.float32),
            jax.ShapeDtypeStruct((T_TYPES, N_NODES, D), jnp.float32),
        ],
    )(x, a_top, a_bot)


_BCAST_DNUMS = lax.GatherDimensionNumbers(
    offset_dims=(), collapsed_slice_dims=(0,), start_index_map=(0,))


def _sc_body(src0, src1, dst, typ, yt, yb, agg_out,
             i4_v, p1_v, t1a_v, t1b_v, g0_v, g1_v, tid_v, dsc_v, nrm_v,
             r0_v, r1_v, ones1_v, zbuf,
             cnt_sh, agg_sh,
             semA0, semA1, semR0, semR1, semS0, semS1, semN0, semN1):
    semA = (semA0, semA1)
    semR = (semR0, semR1)
    semS = (semS0, semS1)
    semN = (semN0, semN1)
    cid = lax.axis_index("c")
    sid = lax.axis_index("s")
    wid = sid * NC + cid

    def _zero16(i, _):
        zbuf[pl.ds(i * L, L)] = jnp.zeros((L,), jnp.float32)
        return 0
    lax.fori_loop(0, ZB // L, _zero16, 0)

    def _ones16(i, _):
        ones1_v[pl.ds(i * L, L)] = jnp.full((L,), 1.0, jnp.float32)
        return 0
    lax.fori_loop(0, C // L, _ones16, 0)

    def _zrow(i, _):
        for j in range(D // L):
            r0_v[0][i, pl.ds(j * L, L)] = jnp.zeros((L,), jnp.float32)
        return 0
    lax.fori_loop(0, C, _zrow, 0)

    @pl.when(sid == 0)
    def _():
        for b in range(TN // ZB):
            pltpu.sync_copy(zbuf, cnt_sh.at[pl.ds(b * ZB, ZB)])

    row0 = sid * ROWS_A

    def _zero_agg_rows(base, nrows):
        off = 0
        while off < nrows:
            n = min(C, nrows - off)
            pltpu.sync_copy(r0_v[0].at[pl.ds(0, n), :],
                            agg_sh.at[pl.ds(base + off, n), :])
            off += n

    @pl.when(sid < NS - 1)
    def _():
        _zero_agg_rows(row0, ROWS_A)

    @pl.when(sid == NS - 1)
    def _():
        _zero_agg_rows(row0, ROWS_LAST)

    plsc.subcore_barrier()

    def _p1_sl(k):
        return pl.ds(sid * E_PER_TILE + k * 2 * C, 2 * C)

    def _p1_fire(k, b):
        @pl.when(k < NK1)
        def _():
            pltpu.async_copy(dst.at[_p1_sl(k)], p1_v[b].at[pl.ds(0, 2 * C)],
                             semA[b])
            pltpu.async_copy(typ.at[_p1_sl(k)], p1_v[b].at[pl.ds(2 * C, 2 * C)],
                             semA[b])

    def _p1_proc(k, b):
        @pl.when(k < NK1)
        def _():
            pltpu.make_async_copy(dst.at[_p1_sl(k)],
                                  p1_v[b].at[pl.ds(0, 2 * C)], semA[b]).wait()
            pltpu.make_async_copy(typ.at[_p1_sl(k)],
                                  p1_v[b].at[pl.ds(2 * C, 2 * C)],
                                  semA[b]).wait()

            for r, t1 in ((0, t1a_v), (1, t1b_v)):
                def _tid16(j, _):
                    t16 = p1_v[b][pl.ds(2 * C + r * C + j * L, L)]
                    d16 = p1_v[b][pl.ds(r * C + j * L, L)]
                    t1[b][pl.ds(j * L, L)] = t16 * N_NODES + d16
                    return 0
                lax.fori_loop(0, C // L, _tid16, 0)
            pltpu.sync_copy(ones1_v, cnt_sh.at[t1a_v[b]], add=True)
            pltpu.sync_copy(ones1_v, cnt_sh.at[t1b_v[b]], add=True)

    _p1_fire(0, 0)
    _p1_fire(1, 1)

    def _p1_pair(i, _):
        k0 = 2 * i
        _p1_proc(k0, 0)
        _p1_fire(k0 + 2, 0)
        _p1_proc(k0 + 1, 1)
        _p1_fire(k0 + 3, 1)
        return 0
    lax.fori_loop(0, (NK1 + 1) // 2, _p1_pair, 0)

    plsc.subcore_barrier()

    def _to_norm(base):
        pltpu.sync_copy(cnt_sh.at[pl.ds(base, ZB)], zbuf)

        def _nrm16(i, _):
            c16 = zbuf[pl.ds(i * L, L)]
            zbuf[pl.ds(i * L, L)] = 1.0 / jnp.maximum(c16, 1.0)
            return 0
        lax.fori_loop(0, ZB // L, _nrm16, 0)
        pltpu.sync_copy(zbuf, cnt_sh.at[pl.ds(base, ZB)])

    _to_norm(sid * ZB)

    @pl.when(sid < TN // ZB - NS)
    def _():
        _to_norm((NS + sid) * ZB)

    plsc.subcore_barrier()

    def _p3_sl(k):
        return pl.ds(wid * EW + k * C, C)

    def _p3_parts(k, b):
        return ((src0, pl.ds(0, C)), (src1, pl.ds(C, C)),
                (dst, pl.ds(2 * C, C)), (typ, pl.ds(3 * C, C)))

    def _p3_fire_idx(k, b):
        @pl.when(k < NK)
        def _():
            for arr, reg in _p3_parts(k, b):
                pltpu.async_copy(arr.at[_p3_sl(k)], i4_v[b].at[reg], semA[b])

    def _p3_fire_rows(k, b):
        @pl.when(k < NK)
        def _():
            for arr, reg in _p3_parts(k, b):
                pltpu.make_async_copy(arr.at[_p3_sl(k)], i4_v[b].at[reg],
                                      semA[b]).wait()

            def _g16(j, _):
                t16 = i4_v[b][pl.ds(3 * C + j * L, L)] * N_NODES
                g0_v[b][pl.ds(j * L, L)] = t16 + i4_v[b][pl.ds(j * L, L)]
                g1_v[b][pl.ds(j * L, L)] = t16 + i4_v[b][pl.ds(C + j * L, L)]
                d16 = i4_v[b][pl.ds(2 * C + j * L, L)]
                dsc_v[b][pl.ds(j * L, L)] = d16
                tid_v[b][pl.ds(j * L, L)] = t16 + d16
                return 0
            lax.fori_loop(0, C // L, _g16, 0)

            pltpu.async_copy(cnt_sh.at[tid_v[b]], nrm_v[b], semN[b])
            pltpu.async_copy(yt.at[g0_v[b]], r0_v[b], semR[b])
            pltpu.async_copy(yb.at[g1_v[b]], r1_v[b], semR[b])

    def _p3_proc(k, b):
        @pl.when(k < NK)
        def _():
            pltpu.make_async_copy(cnt_sh.at[tid_v[b]], nrm_v[b], semN[b]).wait()
            pltpu.make_async_copy(yt.at[g0_v[b]], r0_v[b], semR[b]).wait()
            pltpu.make_async_copy(yb.at[g1_v[b]], r1_v[b], semR[b]).wait()

            def _scale(jj, _):
                nrm16 = nrm_v[b][pl.ds(jj * L, L)]

                def _lane4(i4, _):
                    for u in range(4):
                        i = i4 * 4 + u
                        e = jj * L + i
                        nrm = lax.gather(
                            nrm16, jnp.full((L, 1), i, jnp.int32),
                            _BCAST_DNUMS, slice_sizes=(1,),
                            mode=lax.GatherScatterMode.PROMISE_IN_BOUNDS)
                        for j in range(D // L):
                            sl = pl.ds(j * L, L)
                            r0_v[b][e, sl] = (
                                (r0_v[b][e, sl] + r1_v[b][e, sl]) * nrm)
                    return 0
                lax.fori_loop(0, L // 4, _lane4, 0)
                return 0
            lax.fori_loop(0, C // L, _scale, 0)

            pltpu.async_copy(r0_v[b], agg_sh.at[dsc_v[b]], semS[b], add=True)

    def _p3_wait_scatter(k, b):
        @pl.when((k >= 0) & (k < NK))
        def _():
            pltpu.make_async_copy(r0_v[b], agg_sh.at[dsc_v[b]], semS[b]).wait()

    _p3_fire_idx(0, 0)
    _p3_fire_rows(0, 0)
    _p3_fire_idx(1, 1)

    def _p3_pair(i, _):
        k0 = 2 * i
        _p3_wait_scatter(k0 - 1, 1)
        _p3_fire_rows(k0 + 1, 1)
        _p3_proc(k0, 0)
        _p3_fire_idx(k0 + 2, 0)
        _p3_proc(k0 + 1, 1)
        _p3_fire_idx(k0 + 3, 1)
        _p3_wait_scatter(k0, 0)
        _p3_fire_rows(k0 + 2, 0)
        return 0
    lax.fori_loop(0, (NK + 1) // 2, _p3_pair, 0)

    plsc.subcore_barrier()

    @pl.when(sid < NS - 1)
    def _():
        pltpu.sync_copy(agg_sh.at[pl.ds(row0, ROWS_A), :],
                        agg_out.at[cid, pl.ds(row0, ROWS_A), :])

    @pl.when(sid == NS - 1)
    def _():
        pltpu.sync_copy(agg_sh.at[pl.ds(row0, ROWS_LAST), :],
                        agg_out.at[cid, pl.ds(row0, ROWS_LAST), :])


def _sc_scatter(src0, src1, dst, typ, yt, yb):
    mesh = plsc.VectorSubcoreMesh(core_axis_name="c", subcore_axis_name="s",
                                  num_cores=NC, num_subcores=NS)
    f = pl.kernel(
        _sc_body,
        out_type=jax.ShapeDtypeStruct((NC, N_NODES, D), jnp.float32),
        mesh=mesh,
        scratch_types=[
            [pltpu.VMEM((4 * C,), jnp.int32)] * 2,
            [pltpu.VMEM((P1R * 4 * C,), jnp.int32)] * 2,
            [pltpu.VMEM((C,), jnp.int32)] * 2,
            [pltpu.VMEM((C,), jnp.int32)] * 2,
            [pltpu.VMEM((C,), jnp.int32)] * 2,
            [pltpu.VMEM((C,), jnp.int32)] * 2,
            [pltpu.VMEM((C,), jnp.int32)] * 2,
            [pltpu.VMEM((C,), jnp.int32)] * 2,
            [pltpu.VMEM((C,), jnp.float32)] * 2,
            [pltpu.VMEM((C, D), jnp.float32)] * 2,
            [pltpu.VMEM((C, D), jnp.float32)] * 2,
            pltpu.VMEM((C,), jnp.float32),
            pltpu.VMEM((ZB,), jnp.float32),
            pltpu.VMEM_SHARED((TN,), jnp.float32),
            pltpu.VMEM_SHARED((N_NODES, D), jnp.float32),
            pltpu.SemaphoreType.DMA,
            pltpu.SemaphoreType.DMA,
            pltpu.SemaphoreType.DMA,
            pltpu.SemaphoreType.DMA,
            pltpu.SemaphoreType.DMA,
            pltpu.SemaphoreType.DMA,
            pltpu.SemaphoreType.DMA,
            pltpu.SemaphoreType.DMA,
        ],
    )
    return f(src0, src1, dst, typ, yt, yb)


_BN2 = 2000


def _out_body(x_ref, cwt_ref, cb_ref, a0_ref, a1_ref, o_ref):
    o_ref[...] = (jnp.dot(x_ref[...], cwt_ref[...],
                          preferred_element_type=jnp.float32)
                  + cb_ref[...] + a0_ref[...] + a1_ref[...])


def _combine(x, cwt, cb, a0, a1):
    grid = (N_NODES // _BN2,)
    return pl.pallas_call(
        _out_body,
        grid=grid,
        in_specs=[
            pl.BlockSpec((_BN2, D), lambda i: (i, 0)),
            pl.BlockSpec((D, D), lambda i: (0, 0)),
            pl.BlockSpec((1, D), lambda i: (0, 0)),
            pl.BlockSpec((_BN2, D), lambda i: (i, 0)),
            pl.BlockSpec((_BN2, D), lambda i: (i, 0)),
        ],
        out_specs=pl.BlockSpec((_BN2, D), lambda i: (i, 0)),
        out_shape=jax.ShapeDtypeStruct((N_NODES, D), jnp.float32),
    )(x, cwt, cb, a0, a1)


@jax.jit
def kernel(x, hyperedge_index_2, hyperedge_type_2, A_2, C_w, C_b):
    src = hyperedge_index_2[0]
    src0 = src[0::2]
    src1 = src[1::2]
    dst = hyperedge_index_2[1][0::2]
    a_top = A_2[:, :D, :]
    a_bot = A_2[:, D:, :]

    yt, yb = _y_tables(x, a_top, a_bot)
    yt = yt.reshape(T_TYPES * N_NODES, D)
    yb = yb.reshape(T_TYPES * N_NODES, D)

    agg = _sc_scatter(src0, src1, dst, hyperedge_type_2, yt, yb)

    return _combine(x, C_w.T, C_b.reshape(1, D), agg[0], agg[1])
